# Initial kernel scaffold; baseline (speedup 1.0000x reference)
import jax
import jax.numpy as jnp
from jax.experimental import pallas as pl


def kernel(features, masks, w1, b1, w2, b2):
    raise NotImplementedError("write your pallas kernel here")


# speedup vs baseline: 11.3076x; 11.3076x over previous
from functools import lru_cache

import numpy as np
import jax
import jax.numpy as jnp
from jax import lax
from jax.experimental import pallas as pl
from jax.experimental.pallas import tpu as pltpu

_TEMPERATURE = 0.07
_NA = 256
_NN = 512
_HW = 1024
_B = 16


def _bits_fn():
    skey = jax.random.key(42)
    his, los = [], []
    for b in range(_B):
        ka, kp, kn = jax.random.split(jax.random.fold_in(skey, b), 3)
        for k, n in ((ka, _NA), (kp, _NA), (kn, _NN)):
            hk, lk = jax.random.split(k)
            his.append(jax.random.bits(hk, (n,), jnp.uint32))
            los.append(jax.random.bits(lk, (n,), jnp.uint32))
    hi = jnp.concatenate(his).reshape(_B, 1, _HW)
    lo = jnp.concatenate(los).reshape(_B, 1, _HW)
    f = lambda x: x.astype(jnp.float32)
    return f(hi >> 16), f(hi & 0xFFFF), f(lo >> 16), f(lo & 0xFFFF)


@lru_cache(maxsize=1)
def _rand_bit_halves():
    return tuple(np.asarray(x) for x in jax.jit(_bits_fn)())


_rand_bit_halves()


@lru_cache(maxsize=1)
def _lt_strict():
    return np.tril(np.ones((_HW, _HW), np.float32), -1)


def _mod(a, s):
    q = jnp.floor(a / s)
    r = a - q * s
    r = jnp.where(r < 0.0, r + s, r)
    r = jnp.where(r >= s, r - s, r)
    return r


def _loss_kernel(x_ref, mc_ref, hh_ref, hl_ref, lh_ref, ll_ref,
                 w1_ref, b1_ref, w2_ref, b2_ref, lt_ref, out_ref, acc_ref):
    b = pl.program_id(0)

    @pl.when(b == 0)
    def _():
        acc_ref[0] = 0.0
        acc_ref[1] = 0.0

    x = x_ref[0]
    mcol = mc_ref[0]

    h = jnp.dot(w1_ref[...], x, preferred_element_type=jnp.float32) + b1_ref[...]
    h = 0.5 * h * (1.0 + lax.erf(h * np.float32(1.0 / np.sqrt(2.0))))
    p = jnp.dot(w2_ref[...], h, preferred_element_type=jnp.float32) + b2_ref[...]
    nrm = jnp.sqrt(jnp.sum(p * p, axis=0, keepdims=True))
    p = p / jnp.maximum(nrm, 1e-12)

    cond_f = (mcol > 0.5).astype(jnp.float32)
    cond_b = 1.0 - cond_f
    num_f = jnp.sum(cond_f)
    num_b = np.float32(_HW) - num_f
    iota_col = lax.broadcasted_iota(jnp.int32, (_HW, 1), 0).astype(jnp.float32)
    rank_f = jnp.dot(lt_ref[...], cond_f, preferred_element_type=jnp.float32)
    rank_b = iota_col - rank_f

    col = lax.broadcasted_iota(jnp.int32, (1, _HW), 1).astype(jnp.float32)
    is_fg = col < np.float32(2 * _NA)
    s = jnp.where(is_fg, jnp.maximum(num_f, 1.0), jnp.maximum(num_b, 1.0))

    m65536 = _mod(jnp.full((1, _HW), 65536.0, jnp.float32), s)
    him = _mod(_mod(hh_ref[0], s) * m65536 + _mod(hl_ref[0], s), s)
    lom = _mod(_mod(lh_ref[0], s) * m65536 + _mod(ll_ref[0], s), s)
    mult = _mod(m65536 * m65536, s)
    d = _mod(him * mult + lom, s)

    eq_f = (rank_f == d).astype(jnp.float32) * cond_f
    eq_b = (rank_b == d).astype(jnp.float32) * cond_b
    onehot0 = (iota_col == 0.0).astype(jnp.float32) * jnp.ones((1, _HW), jnp.float32)
    eq_b = jnp.where(num_b > 0.0, eq_b, onehot0)
    eq = jnp.where(is_fg, eq_f, eq_b)

    g = jnp.dot(p, eq, preferred_element_type=jnp.float32)
    a = g[:, :_NA]
    pp = g[:, _NA:2 * _NA]
    n = g[:, 2 * _NA:]

    inv_t = np.float32(1.0 / _TEMPERATURE)
    pos = jnp.sum(a * pp, axis=0, keepdims=True) * inv_t
    negt = lax.dot_general(n, a, (((0,), (0,)), ((), ())),
                           preferred_element_type=jnp.float32) * inv_t
    m = jnp.maximum(jnp.max(negt, axis=0, keepdims=True), pos)
    se = jnp.sum(jnp.exp(negt - m), axis=0, keepdims=True) + jnp.exp(pos - m)
    ce = jnp.mean(m + jnp.log(se) - pos)

    arow = jnp.sum(eq[:, :_NA], axis=1, keepdims=True)
    asum = jnp.sum(iota_col * arow)
    valid = (asum > 0.0).astype(jnp.float32)

    acc_ref[0] += valid * ce
    acc_ref[1] += valid

    @pl.when(b == _B - 1)
    def _():
        out_ref[0, 0] = acc_ref[0] / jnp.maximum(acc_ref[1], 1.0)


def kernel(features, masks, w1, b1, w2, b2):
    x = features.reshape(_B, features.shape[1], _HW)
    mcol = masks.reshape(_B, _HW, 1)
    hh, hl, lh, ll = _rand_bit_halves()
    lt = _lt_strict()
    b1c = b1.reshape(-1, 1)
    b2c = b2.reshape(-1, 1)

    bits_spec = pl.BlockSpec((1, 1, _HW), lambda b: (b, 0, 0))
    full = lambda *shape: pl.BlockSpec(shape, lambda b: (0,) * len(shape))

    out = pl.pallas_call(
        _loss_kernel,
        grid=(_B,),
        in_specs=[
            pl.BlockSpec((1, x.shape[1], _HW), lambda b: (b, 0, 0)),
            pl.BlockSpec((1, _HW, 1), lambda b: (b, 0, 0)),
            bits_spec, bits_spec, bits_spec, bits_spec,
            full(*w1.shape),
            full(*b1c.shape),
            full(*w2.shape),
            full(*b2c.shape),
            full(_HW, _HW),
        ],
        out_specs=pl.BlockSpec(memory_space=pltpu.MemorySpace.SMEM),
        out_shape=jax.ShapeDtypeStruct((1, 1), jnp.float32),
        scratch_shapes=[pltpu.SMEM((2,), jnp.float32)],
        compiler_params=pltpu.CompilerParams(
            dimension_semantics=("arbitrary",)),
    )(x, mcol, hh, hl, lh, ll, w1, b1c, w2, b2c, lt)
    return out[0, 0]


# speedup vs baseline: 12.5665x; 1.1113x over previous
from functools import lru_cache

import numpy as np
import jax
import jax.numpy as jnp
from jax import lax
from jax.experimental import pallas as pl
from jax.experimental.pallas import tpu as pltpu

_TEMPERATURE = 0.07
_NA = 256
_NN = 512
_HW = 1024
_B = 16


def _rotl(x, d):
    return ((x << np.uint32(d)) | (x >> np.uint32(32 - d))).astype(np.uint32)


def _threefry2x32(k0, k1, c0, c1):
    ks0 = np.uint32(k0)
    ks1 = np.uint32(k1)
    ks2 = np.uint32(ks0 ^ ks1 ^ np.uint32(0x1BD11BDA))
    x0 = (c0 + ks0).astype(np.uint32)
    x1 = (c1 + ks1).astype(np.uint32)
    rot = ((13, 15, 26, 6), (17, 29, 16, 24))
    ks = (ks0, ks1, ks2)
    for i in range(5):
        for r in rot[i % 2]:
            x0 = (x0 + x1).astype(np.uint32)
            x1 = _rotl(x1, r)
            x1 = (x1 ^ x0).astype(np.uint32)
        s = i + 1
        x0 = (x0 + ks[s % 3]).astype(np.uint32)
        x1 = (x1 + ks[(s + 1) % 3] + np.uint32(s)).astype(np.uint32)
    return x0, x1


def _tf_child(k, i):
    x0, x1 = _threefry2x32(k[0], k[1], np.zeros_like(i), i.astype(np.uint32))
    return np.stack([x0, x1], axis=-1)


def _tf_bits(k, n):
    i = np.arange(n, dtype=np.uint32)
    x0, x1 = _threefry2x32(k[0], k[1], np.zeros(n, np.uint32), i)
    return (x0 ^ x1).astype(np.uint32)


@lru_cache(maxsize=1)
def _rand_bit_halves():
    skey = np.array([0, 42], dtype=np.uint32)
    his, los = [], []
    for b in range(_B):
        kb = _tf_child(skey, np.uint32(b))
        ka, kp, kn = _tf_child(kb, np.arange(3))
        for k, n in ((ka, _NA), (kp, _NA), (kn, _NN)):
            hk, lk = _tf_child(k, np.arange(2))
            his.append(_tf_bits(hk, n))
            los.append(_tf_bits(lk, n))
    hi = np.concatenate(his).reshape(_B, 1, _HW)
    lo = np.concatenate(los).reshape(_B, 1, _HW)
    f = lambda x: x.astype(np.float32)
    return f(hi >> 16), f(hi & 0xFFFF), f(lo >> 16), f(lo & 0xFFFF)


@lru_cache(maxsize=1)
def _lt_strict():
    return np.tril(np.ones((_HW, _HW), np.float32), -1)


def _mod(a, s):
    q = jnp.floor(a / s)
    r = a - q * s
    r = jnp.where(r < 0.0, r + s, r)
    r = jnp.where(r >= s, r - s, r)
    return r


def _loss_kernel(x_ref, mc_ref, hh_ref, hl_ref, lh_ref, ll_ref,
                 w1_ref, b1_ref, w2_ref, b2_ref, lt_ref, out_ref, acc_ref):
    b = pl.program_id(0)

    @pl.when(b == 0)
    def _():
        acc_ref[0] = 0.0
        acc_ref[1] = 0.0

    x = x_ref[0]
    mcol = mc_ref[0]

    h = jnp.dot(w1_ref[...], x, preferred_element_type=jnp.float32) + b1_ref[...]
    h = 0.5 * h * (1.0 + lax.erf(h * np.float32(1.0 / np.sqrt(2.0))))
    p = jnp.dot(w2_ref[...], h, preferred_element_type=jnp.float32) + b2_ref[...]
    nrm = jnp.sqrt(jnp.sum(p * p, axis=0, keepdims=True))
    p = p / jnp.maximum(nrm, 1e-12)

    cond_f = (mcol > 0.5).astype(jnp.float32)
    num_f = jnp.sum(cond_f)
    num_b = np.float32(_HW) - num_f
    iota_col = lax.broadcasted_iota(jnp.int32, (_HW, 1), 0).astype(jnp.float32)
    rank_f = jnp.dot(lt_ref[...], cond_f, preferred_element_type=jnp.float32)
    rank_b = iota_col - rank_f

    col = lax.broadcasted_iota(jnp.int32, (1, _HW), 1).astype(jnp.float32)
    is_fg = col < np.float32(2 * _NA)
    s = jnp.where(is_fg, jnp.maximum(num_f, 1.0), jnp.maximum(num_b, 1.0))

    m65536 = _mod(jnp.full((1, _HW), 65536.0, jnp.float32), s)
    him = _mod(_mod(hh_ref[0], s) * m65536 + _mod(hl_ref[0], s), s)
    lom = _mod(_mod(lh_ref[0], s) * m65536 + _mod(ll_ref[0], s), s)
    mult = _mod(m65536 * m65536, s)
    d = _mod(him * mult + lom, s)

    r_key = jnp.where(cond_f > 0.0, rank_f, rank_b + np.float32(2048.0))
    d_key = jnp.where(is_fg, d,
                      jnp.where(num_b > 0.0, d + np.float32(2048.0), 0.0))
    eq = (r_key == d_key).astype(jnp.float32)

    g = jnp.dot(p, eq, preferred_element_type=jnp.float32)
    a = g[:, :_NA]
    pp = g[:, _NA:2 * _NA]
    n = g[:, 2 * _NA:]

    inv_t = np.float32(1.0 / _TEMPERATURE)
    pos = jnp.sum(a * pp, axis=0, keepdims=True) * inv_t
    negt = lax.dot_general(n, a, (((0,), (0,)), ((), ())),
                           preferred_element_type=jnp.float32) * inv_t
    m = jnp.maximum(jnp.max(negt, axis=0, keepdims=True), pos)
    se = jnp.sum(jnp.exp(negt - m), axis=0, keepdims=True) + jnp.exp(pos - m)
    ce = jnp.mean(m + jnp.log(se) - pos)

    arow = jnp.sum(eq[:, :_NA], axis=1, keepdims=True)
    asum = jnp.sum(iota_col * arow)
    valid = (asum > 0.0).astype(jnp.float32)

    acc_ref[0] += valid * ce
    acc_ref[1] += valid

    @pl.when(b == _B - 1)
    def _():
        out_ref[0, 0] = acc_ref[0] / jnp.maximum(acc_ref[1], 1.0)


def kernel(features, masks, w1, b1, w2, b2):
    x = features.reshape(_B, features.shape[1], _HW)
    mcol = masks.reshape(_B, _HW, 1)
    hh, hl, lh, ll = _rand_bit_halves()
    lt = _lt_strict()
    b1c = b1.reshape(-1, 1)
    b2c = b2.reshape(-1, 1)

    bits_spec = pl.BlockSpec((1, 1, _HW), lambda b: (b, 0, 0))
    full = lambda *shape: pl.BlockSpec(shape, lambda b: (0,) * len(shape))

    out = pl.pallas_call(
        _loss_kernel,
        grid=(_B,),
        in_specs=[
            pl.BlockSpec((1, x.shape[1], _HW), lambda b: (b, 0, 0)),
            pl.BlockSpec((1, _HW, 1), lambda b: (b, 0, 0)),
            bits_spec, bits_spec, bits_spec, bits_spec,
            full(*w1.shape),
            full(*b1c.shape),
            full(*w2.shape),
            full(*b2c.shape),
            full(_HW, _HW),
        ],
        out_specs=pl.BlockSpec(memory_space=pltpu.MemorySpace.SMEM),
        out_shape=jax.ShapeDtypeStruct((1, 1), jnp.float32),
        scratch_shapes=[pltpu.SMEM((2,), jnp.float32)],
        compiler_params=pltpu.CompilerParams(
            dimension_semantics=("arbitrary",)),
    )(x, mcol, hh, hl, lh, ll, w1, b1c, w2, b2c, lt)
    return out[0, 0]


# speedup vs baseline: 14.0011x; 1.1142x over previous
from functools import lru_cache

import numpy as np
import jax
import jax.numpy as jnp
from jax import lax
from jax.experimental import pallas as pl
from jax.experimental.pallas import tpu as pltpu
from jax.experimental.pallas import tpu_sc as plsc

_TEMPERATURE = 0.07
_NA = 256
_NN = 512
_HW = 1024
_B = 16
_NCHUNK = _HW // 16


def _rotl(x, d):
    return ((x << np.uint32(d)) | (x >> np.uint32(32 - d))).astype(np.uint32)


def _threefry2x32(k0, k1, c0, c1):
    ks0 = np.uint32(k0)
    ks1 = np.uint32(k1)
    ks2 = np.uint32(ks0 ^ ks1 ^ np.uint32(0x1BD11BDA))
    x0 = (c0 + ks0).astype(np.uint32)
    x1 = (c1 + ks1).astype(np.uint32)
    rot = ((13, 15, 26, 6), (17, 29, 16, 24))
    ks = (ks0, ks1, ks2)
    for i in range(5):
        for r in rot[i % 2]:
            x0 = (x0 + x1).astype(np.uint32)
            x1 = _rotl(x1, r)
            x1 = (x1 ^ x0).astype(np.uint32)
        s = i + 1
        x0 = (x0 + ks[s % 3]).astype(np.uint32)
        x1 = (x1 + ks[(s + 1) % 3] + np.uint32(s)).astype(np.uint32)
    return x0, x1


def _tf_child(k, i):
    x0, x1 = _threefry2x32(k[0], k[1], np.zeros_like(i), i.astype(np.uint32))
    return np.stack([x0, x1], axis=-1)


def _tf_bits(k, n):
    i = np.arange(n, dtype=np.uint32)
    x0, x1 = _threefry2x32(k[0], k[1], np.zeros(n, np.uint32), i)
    return (x0 ^ x1).astype(np.uint32)


@lru_cache(maxsize=1)
def _rand_bits():
    skey = np.array([0, 42], dtype=np.uint32)
    his, los = [], []
    for b in range(_B):
        kb = _tf_child(skey, np.uint32(b))
        ka, kp, kn = _tf_child(kb, np.arange(3))
        for k, n in ((ka, _NA), (kp, _NA), (kn, _NN)):
            hk, lk = _tf_child(k, np.arange(2))
            his.append(_tf_bits(hk, n))
            los.append(_tf_bits(lk, n))
    hi = np.concatenate(his).reshape(2 * _B, _NN)
    lo = np.concatenate(los).reshape(2 * _B, _NN)
    return hi, lo


def _sample_kernel(mask_hbm, hi_hbm, lo_hbm, out_hbm,
                   mask_v, hi_v, lo_v, cand_v, idx_v):
    wid = lax.axis_index("s") * 2 + lax.axis_index("c")
    b = wid // 2
    is_fg = (wid % 2) == 0

    pltpu.sync_copy(mask_hbm.at[b], mask_v)
    pltpu.sync_copy(hi_hbm.at[wid], hi_v)
    pltpu.sync_copy(lo_hbm.at[wid], lo_v)

    isfg_v = jnp.full((16,), is_fg, jnp.bool_)

    def body1(j, nf):
        m = mask_v[pl.ds(j * 16, 16)]
        cf = m > 0.5
        ci = cf.astype(jnp.int32)
        inc = plsc.cumsum(ci)
        pix = lax.iota(jnp.int32, 16) + jnp.full((16,), j * 16, jnp.int32)
        fg_rank = (inc - ci) + jnp.full((16,), nf, jnp.int32)
        bg_rank = pix - fg_rank
        rank = jnp.where(isfg_v, fg_rank, bg_rank)
        sel = jnp.logical_xor(cf, jnp.logical_not(isfg_v))
        plsc.store_scatter(cand_v, [rank], pix, mask=sel)
        return nf + jnp.sum(ci)

    num_f = lax.fori_loop(0, _NCHUNK, body1, jnp.int32(0))
    count = jnp.where(is_fg, num_f, _HW - num_f)
    span_v = jnp.maximum(jnp.full((16,), count, jnp.int32),
                         jnp.full((16,), 1, jnp.int32)).astype(jnp.uint32)
    empty_v = jnp.full((16,), count == 0, jnp.bool_)
    zero_v = jnp.full((16,), 0, jnp.int32)

    mult_v = jnp.full((16,), 65536, jnp.uint32) % span_v
    mult_v = (mult_v * mult_v) % span_v

    def body2(j, carry):
        hi = hi_v[pl.ds(j * 16, 16)]
        lo = lo_v[pl.ds(j * 16, 16)]
        d = ((hi % span_v) * mult_v + lo % span_v) % span_v
        g = plsc.load_gather(cand_v, [d.astype(jnp.int32)])
        idx_v[pl.ds(j * 16, 16)] = jnp.where(empty_v, zero_v, g)
        return carry

    lax.fori_loop(0, _NN // 16, body2, jnp.int32(0))
    pltpu.sync_copy(idx_v, out_hbm.at[wid])


def _sample_indices(masks):
    hi, lo = _rand_bits()
    k = pl.kernel(
        _sample_kernel,
        mesh=plsc.VectorSubcoreMesh(core_axis_name="c", subcore_axis_name="s"),
        out_type=jax.ShapeDtypeStruct((2 * _B, _NN), jnp.int32),
        scratch_types=[
            pltpu.VMEM((_HW,), jnp.float32),
            pltpu.VMEM((_NN,), jnp.uint32),
            pltpu.VMEM((_NN,), jnp.uint32),
            pltpu.VMEM((_HW,), jnp.int32),
            pltpu.VMEM((_NN,), jnp.int32),
        ],
        compiler_params=pltpu.CompilerParams(needs_layout_passes=False),
    )
    return k(masks.reshape(_B, _HW), jnp.asarray(hi), jnp.asarray(lo))


def _loss_kernel(x_ref, idx_ref, w1_ref, b1_ref, w2_ref, b2_ref,
                 out_ref, acc_ref):
    b = pl.program_id(0)

    @pl.when(b == 0)
    def _():
        acc_ref[0] = 0.0
        acc_ref[1] = 0.0

    x = x_ref[0]
    idx = idx_ref[0].astype(jnp.float32)

    h = jnp.dot(w1_ref[...], x, preferred_element_type=jnp.float32) + b1_ref[...]
    h = 0.5 * h * (1.0 + lax.erf(h * np.float32(1.0 / np.sqrt(2.0))))
    p = jnp.dot(w2_ref[...], h, preferred_element_type=jnp.float32) + b2_ref[...]
    nrm = jnp.sqrt(jnp.sum(p * p, axis=0, keepdims=True))
    p = p / jnp.maximum(nrm, 1e-12)

    iota_col = lax.broadcasted_iota(jnp.int32, (_HW, 1), 0).astype(jnp.float32)
    eq = (iota_col == idx).astype(jnp.float32)
    g = jnp.dot(p, eq, preferred_element_type=jnp.float32)
    a = g[:, :_NA]
    pp = g[:, _NA:2 * _NA]
    n = g[:, 2 * _NA:]

    inv_t = np.float32(1.0 / _TEMPERATURE)
    pos = jnp.sum(a * pp, axis=0, keepdims=True) * inv_t
    negt = lax.dot_general(n, a, (((0,), (0,)), ((), ())),
                           preferred_element_type=jnp.float32) * inv_t
    m = jnp.maximum(jnp.max(negt, axis=0, keepdims=True), pos)
    se = jnp.sum(jnp.exp(negt - m), axis=0, keepdims=True) + jnp.exp(pos - m)
    ce = jnp.mean(m + jnp.log(se) - pos)

    valid = (jnp.sum(idx[:, :_NA]) > 0.0).astype(jnp.float32)

    acc_ref[0] += valid * ce
    acc_ref[1] += valid

    @pl.when(b == _B - 1)
    def _():
        out_ref[0, 0] = acc_ref[0] / jnp.maximum(acc_ref[1], 1.0)


def kernel(features, masks, w1, b1, w2, b2):
    x = features.reshape(_B, features.shape[1], _HW)
    idx = _sample_indices(masks).reshape(_B, 1, _HW)
    b1c = b1.reshape(-1, 1)
    b2c = b2.reshape(-1, 1)

    full = lambda *shape: pl.BlockSpec(shape, lambda b: (0,) * len(shape))

    out = pl.pallas_call(
        _loss_kernel,
        grid=(_B,),
        in_specs=[
            pl.BlockSpec((1, x.shape[1], _HW), lambda b: (b, 0, 0)),
            pl.BlockSpec((1, 1, _HW), lambda b: (b, 0, 0)),
            full(*w1.shape),
            full(*b1c.shape),
            full(*w2.shape),
            full(*b2c.shape),
        ],
        out_specs=pl.BlockSpec(memory_space=pltpu.MemorySpace.SMEM),
        out_shape=jax.ShapeDtypeStruct((1, 1), jnp.float32),
        scratch_shapes=[pltpu.SMEM((2,), jnp.float32)],
        compiler_params=pltpu.CompilerParams(
            dimension_semantics=("arbitrary",)),
    )(x, idx, w1, b1c, w2, b2c)
    return out[0, 0]


# speedup vs baseline: 14.6519x; 1.0465x over previous
from functools import lru_cache

import numpy as np
import jax
import jax.numpy as jnp
from jax import lax
from jax.experimental import pallas as pl
from jax.experimental.pallas import tpu as pltpu
from jax.experimental.pallas import tpu_sc as plsc

_TEMPERATURE = 0.07
_NA = 256
_NN = 512
_HW = 1024
_B = 16
_NCHUNK = _HW // 16
_PER_STEP = 2


def _rotl(x, d):
    return ((x << np.uint32(d)) | (x >> np.uint32(32 - d))).astype(np.uint32)


def _threefry2x32(k0, k1, c0, c1):
    ks0 = np.uint32(k0)
    ks1 = np.uint32(k1)
    ks2 = np.uint32(ks0 ^ ks1 ^ np.uint32(0x1BD11BDA))
    x0 = (c0 + ks0).astype(np.uint32)
    x1 = (c1 + ks1).astype(np.uint32)
    rot = ((13, 15, 26, 6), (17, 29, 16, 24))
    ks = (ks0, ks1, ks2)
    for i in range(5):
        for r in rot[i % 2]:
            x0 = (x0 + x1).astype(np.uint32)
            x1 = _rotl(x1, r)
            x1 = (x1 ^ x0).astype(np.uint32)
        s = i + 1
        x0 = (x0 + ks[s % 3]).astype(np.uint32)
        x1 = (x1 + ks[(s + 1) % 3] + np.uint32(s)).astype(np.uint32)
    return x0, x1


def _tf_child(k, i):
    x0, x1 = _threefry2x32(k[0], k[1], np.zeros_like(i), i.astype(np.uint32))
    return np.stack([x0, x1], axis=-1)


def _tf_bits(k, n):
    i = np.arange(n, dtype=np.uint32)
    x0, x1 = _threefry2x32(k[0], k[1], np.zeros(n, np.uint32), i)
    return (x0 ^ x1).astype(np.uint32)


@lru_cache(maxsize=1)
def _rand_bits():
    skey = np.array([0, 42], dtype=np.uint32)
    his, los = [], []
    for b in range(_B):
        kb = _tf_child(skey, np.uint32(b))
        ka, kp, kn = _tf_child(kb, np.arange(3))
        for k, n in ((ka, _NA), (kp, _NA), (kn, _NN)):
            hk, lk = _tf_child(k, np.arange(2))
            his.append(_tf_bits(hk, n))
            los.append(_tf_bits(lk, n))
    hi = np.concatenate(his).reshape(2 * _B, _NN)
    lo = np.concatenate(los).reshape(2 * _B, _NN)
    return hi, lo


def _sample_kernel(mask_hbm, hi_hbm, lo_hbm, out_hbm,
                   mask_v, hi_v, lo_v, cand_v, idx_v):
    wid = lax.axis_index("s") * 2 + lax.axis_index("c")
    b = wid // 2
    is_fg = (wid % 2) == 0

    pltpu.sync_copy(mask_hbm.at[b], mask_v)
    pltpu.sync_copy(hi_hbm.at[wid], hi_v)
    pltpu.sync_copy(lo_hbm.at[wid], lo_v)

    isfg_v = jnp.full((16,), is_fg, jnp.bool_)

    def body1(j, nf):
        m = mask_v[pl.ds(j * 16, 16)]
        cf = m > 0.5
        ci = cf.astype(jnp.int32)
        inc = plsc.cumsum(ci)
        pix = lax.iota(jnp.int32, 16) + jnp.full((16,), j * 16, jnp.int32)
        fg_rank = (inc - ci) + jnp.full((16,), nf, jnp.int32)
        bg_rank = pix - fg_rank
        rank = jnp.where(isfg_v, fg_rank, bg_rank)
        sel = jnp.logical_xor(cf, jnp.logical_not(isfg_v))
        plsc.store_scatter(cand_v, [rank], pix, mask=sel)
        return nf + jnp.sum(ci)

    num_f = lax.fori_loop(0, _NCHUNK, body1, jnp.int32(0))
    count = jnp.where(is_fg, num_f, _HW - num_f)
    span_v = jnp.maximum(jnp.full((16,), count, jnp.int32),
                         jnp.full((16,), 1, jnp.int32)).astype(jnp.uint32)
    empty_v = jnp.full((16,), count == 0, jnp.bool_)
    zero_v = jnp.full((16,), 0, jnp.int32)

    mult_v = jnp.full((16,), 65536, jnp.uint32) % span_v
    mult_v = (mult_v * mult_v) % span_v

    def body2(j, carry):
        hi = hi_v[pl.ds(j * 16, 16)]
        lo = lo_v[pl.ds(j * 16, 16)]
        d = ((hi % span_v) * mult_v + lo % span_v) % span_v
        g = plsc.load_gather(cand_v, [d.astype(jnp.int32)])
        idx_v[pl.ds(j * 16, 16)] = jnp.where(empty_v, zero_v, g)
        return carry

    lax.fori_loop(0, _NN // 16, body2, jnp.int32(0))
    pltpu.sync_copy(idx_v, out_hbm.at[wid])


def _sample_indices(masks):
    hi, lo = _rand_bits()
    k = pl.kernel(
        _sample_kernel,
        mesh=plsc.VectorSubcoreMesh(core_axis_name="c", subcore_axis_name="s"),
        out_type=jax.ShapeDtypeStruct((2 * _B, _NN), jnp.int32),
        scratch_types=[
            pltpu.VMEM((_HW,), jnp.float32),
            pltpu.VMEM((_NN,), jnp.uint32),
            pltpu.VMEM((_NN,), jnp.uint32),
            pltpu.VMEM((_HW,), jnp.int32),
            pltpu.VMEM((_NN,), jnp.int32),
        ],
        compiler_params=pltpu.CompilerParams(needs_layout_passes=False),
    )
    return k(masks.reshape(_B, _HW), jnp.asarray(hi), jnp.asarray(lo))


def _loss_kernel(x_ref, idx_ref, w1_ref, b1_ref, w2_ref, b2_ref,
                 out_ref, acc_ref):
    b = pl.program_id(0)

    @pl.when(b == 0)
    def _():
        acc_ref[0] = 0.0
        acc_ref[1] = 0.0

    for t in range(_PER_STEP):
        x = x_ref[t]
        idx = idx_ref[t].astype(jnp.float32)

        h = jnp.dot(w1_ref[...], x, preferred_element_type=jnp.float32) + b1_ref[...]
        h = 0.5 * h * (1.0 + lax.erf(h * np.float32(1.0 / np.sqrt(2.0))))
        p = jnp.dot(w2_ref[...], h, preferred_element_type=jnp.float32) + b2_ref[...]
        nrm = jnp.sqrt(jnp.sum(p * p, axis=0, keepdims=True))
        p = p / jnp.maximum(nrm, 1e-12)

        iota_col = lax.broadcasted_iota(jnp.int32, (_HW, 1), 0).astype(jnp.float32)
        eq = (iota_col == idx).astype(jnp.float32)
        g = jnp.dot(p, eq, preferred_element_type=jnp.float32)
        a = g[:, :_NA]
        pp = g[:, _NA:2 * _NA]
        n = g[:, 2 * _NA:]

        inv_t = np.float32(1.0 / _TEMPERATURE)
        pos = jnp.sum(a * pp, axis=0, keepdims=True) * inv_t
        negt = lax.dot_general(n, a, (((0,), (0,)), ((), ())),
                               preferred_element_type=jnp.float32) * inv_t
        m = jnp.maximum(jnp.max(negt, axis=0, keepdims=True), pos)
        se = jnp.sum(jnp.exp(negt - m), axis=0, keepdims=True) + jnp.exp(pos - m)
        ce = jnp.mean(m + jnp.log(se) - pos)

        valid = (jnp.sum(idx[:, :_NA]) > 0.0).astype(jnp.float32)

        acc_ref[0] += valid * ce
        acc_ref[1] += valid

    @pl.when(b == _B // _PER_STEP - 1)
    def _():
        out_ref[0] = acc_ref[0] / jnp.maximum(acc_ref[1], 1.0)


def kernel(features, masks, w1, b1, w2, b2):
    x = features.reshape(_B, features.shape[1], _HW)
    idx = _sample_indices(masks).reshape(_B, 1, _HW)
    b1c = b1.reshape(-1, 1)
    b2c = b2.reshape(-1, 1)

    full = lambda *shape: pl.BlockSpec(shape, lambda b: (0,) * len(shape))

    out = pl.pallas_call(
        _loss_kernel,
        grid=(_B // _PER_STEP,),
        in_specs=[
            pl.BlockSpec((_PER_STEP, x.shape[1], _HW), lambda b: (b, 0, 0)),
            pl.BlockSpec((_PER_STEP, 1, _HW), lambda b: (b, 0, 0)),
            full(*w1.shape),
            full(*b1c.shape),
            full(*w2.shape),
            full(*b2c.shape),
        ],
        out_specs=pl.BlockSpec(memory_space=pltpu.MemorySpace.SMEM),
        out_shape=jax.ShapeDtypeStruct((1,), jnp.float32),
        scratch_shapes=[pltpu.SMEM((2,), jnp.float32)],
        compiler_params=pltpu.CompilerParams(
            dimension_semantics=("arbitrary",)),
    )(x, idx, w1, b1c, w2, b2c)
    return out.reshape(())


# speedup vs baseline: 14.9084x; 1.0175x over previous
from functools import lru_cache

import numpy as np
import jax
import jax.numpy as jnp
from jax import lax
from jax.experimental import pallas as pl
from jax.experimental.pallas import tpu as pltpu
from jax.experimental.pallas import tpu_sc as plsc

_TEMPERATURE = 0.07
_NA = 256
_NN = 512
_HW = 1024
_B = 16
_NCHUNK = _HW // 16
_PER_STEP = 4


def _rotl(x, d):
    return ((x << np.uint32(d)) | (x >> np.uint32(32 - d))).astype(np.uint32)


def _threefry2x32(k0, k1, c0, c1):
    ks0 = np.uint32(k0)
    ks1 = np.uint32(k1)
    ks2 = np.uint32(ks0 ^ ks1 ^ np.uint32(0x1BD11BDA))
    x0 = (c0 + ks0).astype(np.uint32)
    x1 = (c1 + ks1).astype(np.uint32)
    rot = ((13, 15, 26, 6), (17, 29, 16, 24))
    ks = (ks0, ks1, ks2)
    for i in range(5):
        for r in rot[i % 2]:
            x0 = (x0 + x1).astype(np.uint32)
            x1 = _rotl(x1, r)
            x1 = (x1 ^ x0).astype(np.uint32)
        s = i + 1
        x0 = (x0 + ks[s % 3]).astype(np.uint32)
        x1 = (x1 + ks[(s + 1) % 3] + np.uint32(s)).astype(np.uint32)
    return x0, x1


def _tf_child(k, i):
    x0, x1 = _threefry2x32(k[0], k[1], np.zeros_like(i), i.astype(np.uint32))
    return np.stack([x0, x1], axis=-1)


def _tf_bits(k, n):
    i = np.arange(n, dtype=np.uint32)
    x0, x1 = _threefry2x32(k[0], k[1], np.zeros(n, np.uint32), i)
    return (x0 ^ x1).astype(np.uint32)


@lru_cache(maxsize=1)
def _rand_bits():
    skey = np.array([0, 42], dtype=np.uint32)
    his, los = [], []
    for b in range(_B):
        kb = _tf_child(skey, np.uint32(b))
        ka, kp, kn = _tf_child(kb, np.arange(3))
        for k, n in ((ka, _NA), (kp, _NA), (kn, _NN)):
            hk, lk = _tf_child(k, np.arange(2))
            his.append(_tf_bits(hk, n))
            los.append(_tf_bits(lk, n))
    hi = np.concatenate(his).reshape(2 * _B, _NN)
    lo = np.concatenate(los).reshape(2 * _B, _NN)
    return hi, lo


def _sample_kernel(mask_hbm, hi_hbm, lo_hbm, out_hbm,
                   mask_v, hi_v, lo_v, cand_v, idx_v):
    wid = lax.axis_index("s") * 2 + lax.axis_index("c")
    b = wid // 2
    is_fg = (wid % 2) == 0

    pltpu.sync_copy(mask_hbm.at[b], mask_v)
    pltpu.sync_copy(hi_hbm.at[wid], hi_v)
    pltpu.sync_copy(lo_hbm.at[wid], lo_v)

    isfg_v = jnp.full((16,), is_fg, jnp.bool_)

    def body1(j, nf):
        m = mask_v[pl.ds(j * 16, 16)]
        cf = m > 0.5
        ci = cf.astype(jnp.int32)
        inc = plsc.cumsum(ci)
        pix = lax.iota(jnp.int32, 16) + jnp.full((16,), j * 16, jnp.int32)
        fg_rank = (inc - ci) + jnp.full((16,), nf, jnp.int32)
        bg_rank = pix - fg_rank
        rank = jnp.where(isfg_v, fg_rank, bg_rank)
        sel = jnp.logical_xor(cf, jnp.logical_not(isfg_v))
        plsc.store_scatter(cand_v, [rank], pix, mask=sel)
        return nf + jnp.sum(ci)

    num_f = lax.fori_loop(0, _NCHUNK, body1, jnp.int32(0))
    count = jnp.where(is_fg, num_f, _HW - num_f)
    span_v = jnp.maximum(jnp.full((16,), count, jnp.int32),
                         jnp.full((16,), 1, jnp.int32)).astype(jnp.uint32)
    empty_v = jnp.full((16,), count == 0, jnp.bool_)
    zero_v = jnp.full((16,), 0, jnp.int32)

    mult_v = jnp.full((16,), 65536, jnp.uint32) % span_v
    mult_v = (mult_v * mult_v) % span_v

    def body2(j, carry):
        hi = hi_v[pl.ds(j * 16, 16)]
        lo = lo_v[pl.ds(j * 16, 16)]
        d = ((hi % span_v) * mult_v + lo % span_v) % span_v
        g = plsc.load_gather(cand_v, [d.astype(jnp.int32)])
        idx_v[pl.ds(j * 16, 16)] = jnp.where(empty_v, zero_v, g)
        return carry

    lax.fori_loop(0, _NN // 16, body2, jnp.int32(0))
    pltpu.sync_copy(idx_v, out_hbm.at[wid])


def _sample_indices(masks):
    hi, lo = _rand_bits()
    k = pl.kernel(
        _sample_kernel,
        mesh=plsc.VectorSubcoreMesh(core_axis_name="c", subcore_axis_name="s"),
        out_type=jax.ShapeDtypeStruct((2 * _B, _NN), jnp.int32),
        scratch_types=[
            pltpu.VMEM((_HW,), jnp.float32),
            pltpu.VMEM((_NN,), jnp.uint32),
            pltpu.VMEM((_NN,), jnp.uint32),
            pltpu.VMEM((_HW,), jnp.int32),
            pltpu.VMEM((_NN,), jnp.int32),
        ],
        compiler_params=pltpu.CompilerParams(needs_layout_passes=False),
    )
    return k(masks.reshape(_B, _HW), jnp.asarray(hi), jnp.asarray(lo))


def _loss_kernel(x_ref, idx_ref, w1_ref, b1_ref, w2_ref, b2_ref,
                 out_ref, acc_ref):
    b = pl.program_id(0)

    @pl.when(b == 0)
    def _():
        acc_ref[0] = 0.0
        acc_ref[1] = 0.0

    for t in range(_PER_STEP):
        x = x_ref[t]
        idx = idx_ref[t].astype(jnp.float32)

        h = jnp.dot(w1_ref[...], x, preferred_element_type=jnp.float32) + b1_ref[...]
        h = 0.5 * h * (1.0 + lax.erf(h * np.float32(1.0 / np.sqrt(2.0))))
        p = jnp.dot(w2_ref[...], h, preferred_element_type=jnp.float32) + b2_ref[...]
        nrm = jnp.sqrt(jnp.sum(p * p, axis=0, keepdims=True))
        p = p / jnp.maximum(nrm, 1e-12)

        iota_col = lax.broadcasted_iota(jnp.int32, (_HW, 1), 0).astype(jnp.float32)
        eq = (iota_col == idx).astype(jnp.float32)
        g = jnp.dot(p, eq, preferred_element_type=jnp.float32)
        a = g[:, :_NA]
        pp = g[:, _NA:2 * _NA]
        n = g[:, 2 * _NA:]

        inv_t = np.float32(1.0 / _TEMPERATURE)
        pos = jnp.sum(a * pp, axis=0, keepdims=True) * inv_t
        negt = lax.dot_general(n, a, (((0,), (0,)), ((), ())),
                               preferred_element_type=jnp.float32) * inv_t
        m = jnp.maximum(jnp.max(negt, axis=0, keepdims=True), pos)
        se = jnp.sum(jnp.exp(negt - m), axis=0, keepdims=True) + jnp.exp(pos - m)
        ce = jnp.mean(m + jnp.log(se) - pos)

        valid = (jnp.sum(idx[:, :_NA]) > 0.0).astype(jnp.float32)

        acc_ref[0] += valid * ce
        acc_ref[1] += valid

    @pl.when(b == _B // _PER_STEP - 1)
    def _():
        out_ref[0] = acc_ref[0] / jnp.maximum(acc_ref[1], 1.0)


def kernel(features, masks, w1, b1, w2, b2):
    x = features.reshape(_B, features.shape[1], _HW)
    idx = _sample_indices(masks).reshape(_B, 1, _HW)
    b1c = b1.reshape(-1, 1)
    b2c = b2.reshape(-1, 1)

    full = lambda *shape: pl.BlockSpec(shape, lambda b: (0,) * len(shape))

    out = pl.pallas_call(
        _loss_kernel,
        grid=(_B // _PER_STEP,),
        in_specs=[
            pl.BlockSpec((_PER_STEP, x.shape[1], _HW), lambda b: (b, 0, 0)),
            pl.BlockSpec((_PER_STEP, 1, _HW), lambda b: (b, 0, 0)),
            full(*w1.shape),
            full(*b1c.shape),
            full(*w2.shape),
            full(*b2c.shape),
        ],
        out_specs=pl.BlockSpec(memory_space=pltpu.MemorySpace.SMEM),
        out_shape=jax.ShapeDtypeStruct((1,), jnp.float32),
        scratch_shapes=[pltpu.SMEM((2,), jnp.float32)],
        compiler_params=pltpu.CompilerParams(
            dimension_semantics=("arbitrary",)),
    )(x, idx, w1, b1c, w2, b2c)
    return out.reshape(())
